# baseline (device time: 272657 ns/iter reference)
import jax
import jax.numpy as jnp
from jax import lax
from jax.experimental import pallas as pl
from jax.experimental.pallas import tpu as pltpu

S = 2048
S_HALF = 1024
H = 32
D = 128
K = H * D
N = 8192
NQ = N // 4
BN = 512
BH = 8
BK = BH * D
NJ = NQ // BN
NK = H // BH


def kernel(O, Wo):
    r0 = jnp.full(
        (1,), 2 * lax.axis_index("y") + lax.axis_index("z"), dtype=jnp.int32
    )

    def body(r_ref, o_ref, w_ref, out_ref, keep, send_x, recv_x,
             x_send_sems, x_recv_sems, y_send_sems, y_recv_sems,
             z1_send_sems, z1_recv_sems, z2_send_sems, z2_recv_sems,
             y2_send_sems, y2_recv_sems, outred_sems):
        b = pl.program_id(0)
        k = pl.program_id(1)
        my_x = lax.axis_index("x")
        my_y = lax.axis_index("y")
        my_z = lax.axis_index("z")
        keep_start = my_x * S_HALF
        send_start = (1 - my_x) * S_HALF
        x_partner = (1 - my_x, my_y, my_z)
        y_partner = (my_x, 1 - my_y, my_z)
        z_partner = (my_x, my_y, 1 - my_z)
        d_partner = (my_x, 1 - my_y, 1 - my_z)
        r = 2 * my_y + my_z
        ry = 2 * (1 - my_y) + my_z
        rz = 2 * my_y + (1 - my_z)
        rd = 2 * (1 - my_y) + (1 - my_z)

        def out_slice(quarter, i):
            return out_ref.at[0, :, pl.ds(quarter * NQ + i * BN, BN)]

        def x_rdma(i):
            return pltpu.make_async_remote_copy(
                src_ref=send_x.at[i],
                dst_ref=recv_x.at[i],
                send_sem=x_send_sems.at[i],
                recv_sem=x_recv_sems.at[i],
                device_id=x_partner,
                device_id_type=pl.DeviceIdType.MESH,
            )

        def y_rdma(i):
            return pltpu.make_async_remote_copy(
                src_ref=keep.at[i],
                dst_ref=out_slice(r, i),
                send_sem=y_send_sems.at[i],
                recv_sem=y_recv_sems.at[i],
                device_id=y_partner,
                device_id_type=pl.DeviceIdType.MESH,
            )

        def y_wait(i):
            return pltpu.make_async_remote_copy(
                src_ref=keep.at[i],
                dst_ref=out_slice(ry, i),
                send_sem=y_send_sems.at[i],
                recv_sem=y_recv_sems.at[i],
                device_id=y_partner,
                device_id_type=pl.DeviceIdType.MESH,
            )

        def z1_rdma(i):
            return pltpu.make_async_remote_copy(
                src_ref=keep.at[i],
                dst_ref=out_slice(r, i),
                send_sem=z1_send_sems.at[i],
                recv_sem=z1_recv_sems.at[i],
                device_id=z_partner,
                device_id_type=pl.DeviceIdType.MESH,
            )

        def z1_wait(i):
            return pltpu.make_async_remote_copy(
                src_ref=keep.at[i],
                dst_ref=out_slice(rz, i),
                send_sem=z1_send_sems.at[i],
                recv_sem=z1_recv_sems.at[i],
                device_id=z_partner,
                device_id_type=pl.DeviceIdType.MESH,
            )

        def z2_rdma(i):
            return pltpu.make_async_remote_copy(
                src_ref=out_slice(ry, i),
                dst_ref=out_slice(ry, i),
                send_sem=z2_send_sems.at[i],
                recv_sem=z2_recv_sems.at[i],
                device_id=z_partner,
                device_id_type=pl.DeviceIdType.MESH,
            )

        def z2_wait(i):
            return pltpu.make_async_remote_copy(
                src_ref=out_slice(rd, i),
                dst_ref=out_slice(rd, i),
                send_sem=z2_send_sems.at[i],
                recv_sem=z2_recv_sems.at[i],
                device_id=z_partner,
                device_id_type=pl.DeviceIdType.MESH,
            )

        def d_rdma(i):
            return pltpu.make_async_remote_copy(
                src_ref=keep.at[i],
                dst_ref=out_slice(r, i),
                send_sem=y2_send_sems.at[i],
                recv_sem=y2_recv_sems.at[i],
                device_id=d_partner,
                device_id_type=pl.DeviceIdType.MESH,
            )

        def d_wait(i):
            return pltpu.make_async_remote_copy(
                src_ref=keep.at[i],
                dst_ref=out_slice(rd, i),
                send_sem=y2_send_sems.at[i],
                recv_sem=y2_recv_sems.at[i],
                device_id=d_partner,
                device_id_type=pl.DeviceIdType.MESH,
            )

        def half_partial(row_start):
            acc = None
            for h in range(BH):
                a = o_ref[0, pl.ds(row_start, S_HALF), h, :]
                w = w_ref[pl.ds(h * D, D), :]
                d = jnp.dot(a, w, preferred_element_type=jnp.float32)
                acc = d if acc is None else acc + d
            return acc

        @pl.when(b < NJ)
        def _():
            prod_send = half_partial(send_start)
            prod_keep = half_partial(keep_start)

            @pl.when(k == 0)
            def _():
                send_x[b] = prod_send
                keep[b] = prod_keep

            @pl.when(k != 0)
            def _():
                send_x[b] += prod_send
                keep[b] += prod_keep

            @pl.when(k == NK - 1)
            def _():
                x_rdma(b).start()

        @pl.when(k == NK - 1)
        def _():
            @pl.when((b >= 1) & (b <= NJ))
            def _():
                c = b - 1
                x_rdma(c).wait_recv()
                keep[c] += recv_x[c]
                y_rdma(c).start()
                z1_rdma(c).start()

                @pl.when(c % 2 == 1)
                def _():
                    d_rdma(c).start()

                pltpu.make_async_copy(
                    keep.at[c], out_slice(r, c), outred_sems.at[c]
                ).start()

            @pl.when((b >= 2) & (b <= NJ + 1))
            def _():
                c2 = b - 2
                y_wait(c2).wait_recv()
                z1_wait(c2).wait_recv()

                @pl.when(c2 % 2 == 0)
                def _():
                    z2_rdma(c2).start()

            @pl.when((b >= 3) & (b <= NJ + 2))
            def _():
                c3 = b - 3

                @pl.when(c3 % 2 == 0)
                def _():
                    z2_wait(c3).wait_recv()

                @pl.when(c3 % 2 == 1)
                def _():
                    d_wait(c3).wait_recv()

            @pl.when(b == NJ + 2)
            def _():
                for i in range(NJ):
                    x_rdma(i).wait_send()
                    y_rdma(i).wait_send()
                    z1_rdma(i).wait_send()
                    if i % 2 == 0:
                        z2_rdma(i).wait_send()
                    else:
                        d_rdma(i).wait_send()
                    pltpu.make_async_copy(
                        keep.at[i], out_slice(r, i), outred_sems.at[i]
                    ).wait()

    grid_spec = pltpu.PrefetchScalarGridSpec(
        num_scalar_prefetch=1,
        grid=(NJ + 3, NK),
        in_specs=[
            pl.BlockSpec(
                (1, S, BH, D),
                lambda b, k, r_ref: (0, 0, k, 0),
            ),
            pl.BlockSpec(
                (BK, BN),
                lambda b, k, r_ref: (
                    k,
                    r_ref[0] * NJ + jnp.minimum(b, NJ - 1),
                ),
            ),
        ],
        out_specs=pl.BlockSpec(memory_space=pl.ANY),
        scratch_shapes=[
            pltpu.VMEM((NJ, S_HALF, BN), jnp.float32),
            pltpu.VMEM((NJ, S_HALF, BN), jnp.float32),
            pltpu.VMEM((NJ, S_HALF, BN), jnp.float32),
            pltpu.SemaphoreType.DMA((NJ,)),
            pltpu.SemaphoreType.DMA((NJ,)),
            pltpu.SemaphoreType.DMA((NJ,)),
            pltpu.SemaphoreType.DMA((NJ,)),
            pltpu.SemaphoreType.DMA((NJ,)),
            pltpu.SemaphoreType.DMA((NJ,)),
            pltpu.SemaphoreType.DMA((NJ,)),
            pltpu.SemaphoreType.DMA((NJ,)),
            pltpu.SemaphoreType.DMA((NJ,)),
            pltpu.SemaphoreType.DMA((NJ,)),
            pltpu.SemaphoreType.DMA((NJ,)),
        ],
    )

    out = pl.pallas_call(
        body,
        grid_spec=grid_spec,
        out_shape=jax.ShapeDtypeStruct((1, S_HALF, N), jnp.float32),
        compiler_params=pltpu.CompilerParams(
            vmem_limit_bytes=60 * 1024 * 1024,
        ),
    )(r0, O, Wo)
    return out


# device time: 227801 ns/iter; 1.1969x vs baseline; 1.1969x over previous
import jax
import jax.numpy as jnp
from jax import lax
from jax.experimental import pallas as pl
from jax.experimental.pallas import tpu as pltpu

S = 2048
S_HALF = 1024
H = 32
D = 128
K = H * D
N = 8192
NQ = N // 4
BN = 512
BH = 8
BK = BH * D
NJ = NQ // BN
NK = H // BH


def kernel(O, Wo):
    r0 = jnp.full(
        (1,), 2 * lax.axis_index("y") + lax.axis_index("z"), dtype=jnp.int32
    )

    def body(r_ref, o_ref, w_ref, out_ref, keep, send_x, recv_x,
             x_send_sems, x_recv_sems, y_send_sems, y_recv_sems,
             z1_send_sems, z1_recv_sems, z2_send_sems, z2_recv_sems,
             y2_send_sems, y2_recv_sems, outred_sems):
        b = pl.program_id(0)
        k = pl.program_id(1)
        my_x = lax.axis_index("x")
        my_y = lax.axis_index("y")
        my_z = lax.axis_index("z")
        keep_start = my_x * S_HALF
        send_start = (1 - my_x) * S_HALF
        x_partner = (1 - my_x, my_y, my_z)
        y_partner = (my_x, 1 - my_y, my_z)
        z_partner = (my_x, my_y, 1 - my_z)
        r = 2 * my_y + my_z
        ry = 2 * (1 - my_y) + my_z
        rz = 2 * my_y + (1 - my_z)
        rd = 2 * (1 - my_y) + (1 - my_z)

        def out_slice(quarter, i):
            return out_ref.at[0, :, pl.ds(quarter * NQ + i * BN, BN)]

        def x_rdma(i):
            return pltpu.make_async_remote_copy(
                src_ref=send_x.at[i],
                dst_ref=recv_x.at[i],
                send_sem=x_send_sems.at[i],
                recv_sem=x_recv_sems.at[i],
                device_id=x_partner,
                device_id_type=pl.DeviceIdType.MESH,
            )

        def y_rdma(i):
            return pltpu.make_async_remote_copy(
                src_ref=keep.at[i],
                dst_ref=out_slice(r, i),
                send_sem=y_send_sems.at[i],
                recv_sem=y_recv_sems.at[i],
                device_id=y_partner,
                device_id_type=pl.DeviceIdType.MESH,
            )

        def y_wait(i):
            return pltpu.make_async_remote_copy(
                src_ref=keep.at[i],
                dst_ref=out_slice(ry, i),
                send_sem=y_send_sems.at[i],
                recv_sem=y_recv_sems.at[i],
                device_id=y_partner,
                device_id_type=pl.DeviceIdType.MESH,
            )

        def z1_rdma(i):
            return pltpu.make_async_remote_copy(
                src_ref=keep.at[i],
                dst_ref=out_slice(r, i),
                send_sem=z1_send_sems.at[i],
                recv_sem=z1_recv_sems.at[i],
                device_id=z_partner,
                device_id_type=pl.DeviceIdType.MESH,
            )

        def z1_wait(i):
            return pltpu.make_async_remote_copy(
                src_ref=keep.at[i],
                dst_ref=out_slice(rz, i),
                send_sem=z1_send_sems.at[i],
                recv_sem=z1_recv_sems.at[i],
                device_id=z_partner,
                device_id_type=pl.DeviceIdType.MESH,
            )

        def z2_rdma(i):
            return pltpu.make_async_remote_copy(
                src_ref=out_slice(ry, i),
                dst_ref=out_slice(ry, i),
                send_sem=z2_send_sems.at[i],
                recv_sem=z2_recv_sems.at[i],
                device_id=z_partner,
                device_id_type=pl.DeviceIdType.MESH,
            )

        def z2_wait(i):
            return pltpu.make_async_remote_copy(
                src_ref=out_slice(rd, i),
                dst_ref=out_slice(rd, i),
                send_sem=z2_send_sems.at[i],
                recv_sem=z2_recv_sems.at[i],
                device_id=z_partner,
                device_id_type=pl.DeviceIdType.MESH,
            )

        def y2_rdma(i):
            return pltpu.make_async_remote_copy(
                src_ref=out_slice(rz, i),
                dst_ref=out_slice(rz, i),
                send_sem=y2_send_sems.at[i],
                recv_sem=y2_recv_sems.at[i],
                device_id=y_partner,
                device_id_type=pl.DeviceIdType.MESH,
            )

        def y2_wait(i):
            return pltpu.make_async_remote_copy(
                src_ref=out_slice(rd, i),
                dst_ref=out_slice(rd, i),
                send_sem=y2_send_sems.at[i],
                recv_sem=y2_recv_sems.at[i],
                device_id=y_partner,
                device_id_type=pl.DeviceIdType.MESH,
            )

        def half_partial(row_start):
            acc = None
            for h in range(BH):
                a = o_ref[0, pl.ds(row_start, S_HALF), h, :]
                w = w_ref[pl.ds(h * D, D), :]
                d = jnp.dot(a, w, preferred_element_type=jnp.float32)
                acc = d if acc is None else acc + d
            return acc

        @pl.when(b < NJ)
        def _():
            prod_send = half_partial(send_start)
            prod_keep = half_partial(keep_start)

            @pl.when(k == 0)
            def _():
                send_x[b] = prod_send
                keep[b] = prod_keep

            @pl.when(k != 0)
            def _():
                send_x[b] += prod_send
                keep[b] += prod_keep

            @pl.when(k == NK - 1)
            def _():
                x_rdma(b).start()

        @pl.when(k == NK - 1)
        def _():
            @pl.when((b >= 1) & (b <= NJ))
            def _():
                c = b - 1
                x_rdma(c).wait_recv()
                keep[c] += recv_x[c]
                y_rdma(c).start()
                z1_rdma(c).start()
                pltpu.make_async_copy(
                    keep.at[c], out_slice(r, c), outred_sems.at[c]
                ).start()

            @pl.when((b >= 2) & (b <= NJ + 1))
            def _():
                c2 = b - 2
                y_wait(c2).wait_recv()
                z1_wait(c2).wait_recv()

                @pl.when(c2 % 2 == 0)
                def _():
                    z2_rdma(c2).start()

                @pl.when(c2 % 2 == 1)
                def _():
                    y2_rdma(c2).start()

            @pl.when((b >= 3) & (b <= NJ + 2))
            def _():
                c3 = b - 3

                @pl.when(c3 % 2 == 0)
                def _():
                    z2_wait(c3).wait_recv()

                @pl.when(c3 % 2 == 1)
                def _():
                    y2_wait(c3).wait_recv()

            @pl.when(b == NJ + 2)
            def _():
                for i in range(NJ):
                    x_rdma(i).wait_send()
                    y_rdma(i).wait_send()
                    z1_rdma(i).wait_send()
                    if i % 2 == 0:
                        z2_rdma(i).wait_send()
                    else:
                        y2_rdma(i).wait_send()
                    pltpu.make_async_copy(
                        keep.at[i], out_slice(r, i), outred_sems.at[i]
                    ).wait()

    grid_spec = pltpu.PrefetchScalarGridSpec(
        num_scalar_prefetch=1,
        grid=(NJ + 3, NK),
        in_specs=[
            pl.BlockSpec(
                (1, S, BH, D),
                lambda b, k, r_ref: (0, 0, k, 0),
            ),
            pl.BlockSpec(
                (BK, BN),
                lambda b, k, r_ref: (
                    k,
                    r_ref[0] * NJ + jnp.minimum(b, NJ - 1),
                ),
            ),
        ],
        out_specs=pl.BlockSpec(memory_space=pl.ANY),
        scratch_shapes=[
            pltpu.VMEM((NJ, S_HALF, BN), jnp.float32),
            pltpu.VMEM((NJ, S_HALF, BN), jnp.float32),
            pltpu.VMEM((NJ, S_HALF, BN), jnp.float32),
            pltpu.SemaphoreType.DMA((NJ,)),
            pltpu.SemaphoreType.DMA((NJ,)),
            pltpu.SemaphoreType.DMA((NJ,)),
            pltpu.SemaphoreType.DMA((NJ,)),
            pltpu.SemaphoreType.DMA((NJ,)),
            pltpu.SemaphoreType.DMA((NJ,)),
            pltpu.SemaphoreType.DMA((NJ,)),
            pltpu.SemaphoreType.DMA((NJ,)),
            pltpu.SemaphoreType.DMA((NJ,)),
            pltpu.SemaphoreType.DMA((NJ,)),
            pltpu.SemaphoreType.DMA((NJ,)),
        ],
    )

    out = pl.pallas_call(
        body,
        grid_spec=grid_spec,
        out_shape=jax.ShapeDtypeStruct((1, S_HALF, N), jnp.float32),
        compiler_params=pltpu.CompilerParams(
            vmem_limit_bytes=60 * 1024 * 1024,
        ),
    )(r0, O, Wo)
    return out
